# Initial kernel scaffold; baseline (speedup 1.0000x reference)
import jax
import jax.numpy as jnp
from jax.experimental import pallas as pl


def kernel(depth_image, rgb_image, key):
    raise NotImplementedError("write your pallas kernel here")


# speedup vs baseline: 29.0394x; 29.0394x over previous
import jax
import jax.numpy as jnp
from jax import lax
from jax.experimental import pallas as pl
from jax.experimental.pallas import tpu as pltpu

H = 512
W = 512
NPTS = 2048
MIN_DEPTH = 0.1
MAX_DEPTH = 2.0
FX = 525.0
FY = 525.0
CX = (W - 1) / 2.0
CY = (H - 1) / 2.0
BIG = 1 << 30


def _fps_body(depth_ref, rgbf_ref, idx_ref, sxyz_ref, srgb_ref, px, py, pz, dist):
    depth = depth_ref[...]
    u = lax.broadcasted_iota(jnp.int32, (H, W), 1).astype(jnp.float32)
    v = lax.broadcasted_iota(jnp.int32, (H, W), 0).astype(jnp.float32)
    x = (u - CX) * depth / FX
    y = (v - CY) * depth / FY
    finite = (depth - depth) == 0.0
    valid = (depth > MIN_DEPTH) & (depth < MAX_DEPTH) & (depth > 0.0) & finite
    px[...] = jnp.where(valid, x, 0.0)
    py[...] = jnp.where(valid, y, 0.0)
    pz[...] = jnp.where(valid, depth, 0.0)
    dist[...] = jnp.where(valid, 1e38, -1e38)

    colio = lax.broadcasted_iota(jnp.int32, (H, W), 1)
    rowio = lax.broadcasted_iota(jnp.int32, (H, 1), 0)
    colio1 = lax.broadcasted_iota(jnp.int32, (1, W), 1)
    colio3 = lax.broadcasted_iota(jnp.int32, (1, 3 * W), 1)

    colcand = jnp.where(valid, colio, BIG)
    rowmin = jnp.min(colcand, axis=1, keepdims=True)
    rcand = jnp.where(rowmin < BIG, rowio, BIG)
    r0 = jnp.min(rcand)
    r0 = jnp.where(r0 < BIG, r0, 0)
    dr0 = depth_ref[pl.ds(r0, 1), :]
    fin0 = (dr0 - dr0) == 0.0
    val0 = (dr0 > MIN_DEPTH) & (dr0 < MAX_DEPTH) & (dr0 > 0.0) & fin0
    c0 = jnp.min(jnp.where(val0, colio1, BIG))
    c0 = jnp.where(c0 < BIG, c0, 0)

    def body(i, rc):
        r, c = rc
        pxr = px[pl.ds(r, 1), :]
        pyr = py[pl.ds(r, 1), :]
        pzr = pz[pl.ds(r, 1), :]
        sel = colio1 == c
        cxs = jnp.sum(jnp.where(sel, pxr, 0.0))
        cys = jnp.sum(jnp.where(sel, pyr, 0.0))
        czs = jnp.sum(jnp.where(sel, pzr, 0.0))
        idx_ref[i] = r * W + c
        sxyz_ref[0, i] = cxs
        sxyz_ref[1, i] = cys
        sxyz_ref[2, i] = czs
        rgbr = rgbf_ref[pl.ds(r, 1), :]
        c3 = c * 3
        srgb_ref[0, i] = jnp.sum(jnp.where(colio3 == c3, rgbr, 0.0))
        srgb_ref[1, i] = jnp.sum(jnp.where(colio3 == c3 + 1, rgbr, 0.0))
        srgb_ref[2, i] = jnp.sum(jnp.where(colio3 == c3 + 2, rgbr, 0.0))
        dx = px[...] - cxs
        dy = py[...] - cys
        dz = pz[...] - czs
        d = dx * dx + dy * dy + dz * dz
        nd = jnp.minimum(dist[...], d)
        dist[...] = nd
        rowmax = jnp.max(nd, axis=1, keepdims=True)
        m = jnp.max(rowmax)
        r2 = jnp.min(jnp.where(rowmax == m, rowio, BIG))
        drow = dist[pl.ds(r2, 1), :]
        c2 = jnp.min(jnp.where(drow == m, colio1, BIG))
        return (r2, c2)

    lax.fori_loop(0, NPTS, body, (r0, c0))


def _assemble_body(sxyz_ref, srgb_ref, out_ref):
    s = sxyz_ref[...]
    rgb = srgb_ref[...] / 255.0
    mn = jnp.min(s, axis=1, keepdims=True)
    centered = s - mn
    mx = jnp.max(centered, axis=1, keepdims=True)
    mx = jnp.where(mx < 1e-8, 1.0, mx)
    out_ref[...] = jnp.concatenate([s, rgb, centered / mx], axis=0)


def kernel(depth_image, rgb_image, key):
    rgbf = rgb_image.reshape(H, 3 * W)
    idx, sxyz, srgb = pl.pallas_call(
        _fps_body,
        out_shape=[
            jax.ShapeDtypeStruct((NPTS,), jnp.int32),
            jax.ShapeDtypeStruct((3, NPTS), jnp.float32),
            jax.ShapeDtypeStruct((3, NPTS), jnp.float32),
        ],
        in_specs=[
            pl.BlockSpec(memory_space=pltpu.VMEM),
            pl.BlockSpec(memory_space=pltpu.VMEM),
        ],
        out_specs=[
            pl.BlockSpec(memory_space=pltpu.SMEM),
            pl.BlockSpec(memory_space=pltpu.SMEM),
            pl.BlockSpec(memory_space=pltpu.SMEM),
        ],
        scratch_shapes=[pltpu.VMEM((H, W), jnp.float32)] * 4,
    )(depth_image, rgbf)
    del idx
    out_t = pl.pallas_call(
        _assemble_body,
        out_shape=jax.ShapeDtypeStruct((9, NPTS), jnp.float32),
    )(sxyz, srgb)
    return out_t.T
